# baseline (speedup 1.0000x reference)
import functools

import jax
import jax.numpy as jnp
from jax.experimental import pallas as pl

_F32 = jnp.float32
_TR = 32


def _silu(z):
    t = 0.5 * z
    return t * jnp.tanh(t) + t


def _prep_kernel(x8_ref, h_ref, wea_ref, web_ref, be0_ref,
                 a_ref, b_ref, p_ref, q_ref, x_ref):
    x = x8_ref[...]
    h = h_ref[...]
    n = x.shape[0]
    a_ref[...] = jnp.dot(h, wea_ref[...], preferred_element_type=_F32)
    b_ref[...] = jnp.dot(h, web_ref[...], preferred_element_type=_F32) \
        + be0_ref[...]
    x3 = x[:, 0:3]
    nx = jnp.sum(x3 * x3, axis=1, keepdims=True)
    ones = jnp.ones((n, 1), _F32)
    zer3 = jnp.zeros((n, 3), _F32)
    p_ref[...] = jnp.concatenate([x3, nx, ones, zer3], axis=1)
    q_ref[...] = jnp.concatenate([-2.0 * x3, ones, nx, zer3], axis=1)
    x_ref[...] = x


def _edge_kernel(a2_ref, a_ref, b_ref, p_ref, q2_ref, x2_ref,
                 gt_ref, g_ref,
                 we1bd_ref, be1d_ref, wx0bd_ref, bx0d_ref,
                 wx1sel_ref, bx1_ref, selw_ref, sel16_ref,
                 we1_ref, be1_ref,
                 magg_ref, shift_ref, *, n, tr, hdim):
    n2 = n // 2
    e2 = tr * n2

    a_part = jnp.broadcast_to(a2_ref[...], (tr, n2, 2 * hdim)).reshape(
        e2, 2 * hdim)
    qt = jnp.broadcast_to(q2_ref[...], (tr, n2, 16)).reshape(e2, 16)

    gt = gt_ref[...]
    g = g_ref[...]

    bt = b_ref[...]
    pt = p_ref[...]
    b2t = jnp.concatenate([bt, bt], axis=1)
    p2t = jnp.concatenate([pt, pt], axis=1)

    p_part = jnp.dot(gt, p2t, preferred_element_type=_F32)
    pq = qt * p_part

    sq16 = jnp.maximum(
        jnp.dot(pq, sel16_ref[...], preferred_element_type=_F32), 0.0)
    gtpq = jnp.concatenate([gt, pq], axis=1)
    bsel = jnp.concatenate([b2t, selw_ref[...]], axis=0)
    bsq_part = jnp.dot(gtpq, bsel, preferred_element_type=_F32)

    bf16 = jnp.bfloat16
    m0 = _silu(a_part + bsq_part).astype(bf16)
    m1 = _silu(jnp.dot(m0, we1bd_ref[...], preferred_element_type=_F32)
               + be1d_ref[...]).astype(bf16)
    av = _silu(jnp.dot(m1, wx0bd_ref[...], preferred_element_type=_F32)
               + bx0d_ref[...]).astype(bf16)
    coef16 = jnp.dot(av, wx1sel_ref[...], preferred_element_type=_F32) \
        + bx1_ref[...]
    scale16 = coef16 / (jnp.sqrt(sq16) + 1.0)

    maggp = jnp.dot(g, m1, preferred_element_type=_F32)
    m0d = _silu(a_ref[...] + bt)
    m1d = _silu(jnp.dot(m0d, we1_ref[...], preferred_element_type=_F32)
                + be1_ref[...])
    magg_ref[...] = maggp[:, :hdim] + maggp[:, hdim:] - m1d

    xt = jnp.broadcast_to(x2_ref[...], (tr, n2, 16)).reshape(e2, 16)
    sxs = jnp.concatenate([scale16 * xt, scale16], axis=1).astype(bf16)
    sump = jnp.dot(g, sxs, preferred_element_type=_F32)
    part1 = sump[:, 0:8] + sump[:, 8:16]
    ssum = sump[:, 16:17] + sump[:, 24:25]
    shift_ref[...] = part1 - pt * ssum


def _update_kernel(h_ref, x8_ref, magg_ref, shift_ref,
                   wh0a_ref, wh0b_ref, bh0_ref, wh1_ref, bh1_ref,
                   hn_ref, xn_ref, *, n):
    inv = _F32(1.0 / (n - 1))
    h = h_ref[...]
    magg = magg_ref[...] * inv
    t = _silu(jnp.dot(h, wh0a_ref[...], preferred_element_type=_F32)
              + jnp.dot(magg, wh0b_ref[...], preferred_element_type=_F32)
              + bh0_ref[...])
    dh = jnp.dot(t, wh1_ref[...], preferred_element_type=_F32) + bh1_ref[...]
    hn_ref[...] = h + dh
    xs = x8_ref[:, 0:3] + shift_ref[:, 0:3] * inv
    xn_ref[...] = jnp.concatenate([xs, jnp.zeros((n, 5), _F32)], axis=1)


def _update_head_kernel(h_ref, x8_ref, magg_ref, shift_ref, pos8_ref,
                        wh0a_ref, wh0b_ref, bh0_ref, wh1_ref, bh1_ref,
                        wf_ref, bf_ref,
                        hout_ref, vec_ref, *, n):
    inv = _F32(1.0 / (n - 1))
    h = h_ref[...]
    magg = magg_ref[...] * inv
    t = _silu(jnp.dot(h, wh0a_ref[...], preferred_element_type=_F32)
              + jnp.dot(magg, wh0b_ref[...], preferred_element_type=_F32)
              + bh0_ref[...])
    dh = jnp.dot(t, wh1_ref[...], preferred_element_type=_F32) + bh1_ref[...]
    h2 = h + dh
    xs = x8_ref[:, 0:3] + shift_ref[:, 0:3] * inv
    vec3 = xs - pos8_ref[:, 0:3]
    vec_ref[...] = jnp.concatenate([vec3, jnp.zeros((n, 5), _F32)], axis=1)
    z = h2 - jnp.max(h2, axis=1, keepdims=True)
    ez = jnp.exp(z)
    sm = ez / jnp.sum(ez, axis=1, keepdims=True)
    hout_ref[...] = jnp.dot(sm, wf_ref[...], preferred_element_type=_F32) \
        + bf_ref[...]


def kernel(positions, features, We0, be0, We1, be1, Wx0, bx0, Wx1, bx1,
           Wh0, bh0, Wh1, bh1, Wf, bf):
    n = positions.shape[0]
    hdim = features.shape[-1]
    nb = We0.shape[0]
    tr = _TR
    nt = n // tr
    n2 = n // 2
    e2 = tr * n2

    h = features[:, 0, :].astype(_F32)
    x8 = jnp.pad(positions[:, 0, :].astype(_F32), ((0, 0), (0, 5)))
    pos8 = x8

    recv_of_row = jnp.arange(e2, dtype=jnp.int32) // n2
    gt_const = (recv_of_row[:, None]
                == jnp.arange(tr, dtype=jnp.int32)[None, :]).astype(_F32)
    g_const = gt_const.T.astype(jnp.bfloat16)
    sel16 = ((jnp.arange(16)[:, None] // 8)
             == (jnp.arange(16)[None, :] // 8)).astype(_F32)

    prep_call = pl.pallas_call(
        _prep_kernel,
        out_shape=[
            jax.ShapeDtypeStruct((n, hdim), _F32),
            jax.ShapeDtypeStruct((n, hdim), _F32),
            jax.ShapeDtypeStruct((n, 8), _F32),
            jax.ShapeDtypeStruct((n, 8), _F32),
            jax.ShapeDtypeStruct((n, 8), _F32),
        ],
    )

    full = lambda t: (0, 0)
    tile = lambda t: (t, 0)
    edge_call = pl.pallas_call(
        functools.partial(_edge_kernel, n=n, tr=tr, hdim=hdim),
        grid=(nt,),
        in_specs=[
            pl.BlockSpec((1, n2, 2 * hdim), lambda t: (0, 0, 0)),
            pl.BlockSpec((tr, hdim), tile),
            pl.BlockSpec((tr, hdim), tile),
            pl.BlockSpec((tr, 8), tile),
            pl.BlockSpec((1, n2, 16), lambda t: (0, 0, 0)),
            pl.BlockSpec((1, n2, 16), lambda t: (0, 0, 0)),
            pl.BlockSpec((e2, tr), full),
            pl.BlockSpec((tr, e2), full),
            pl.BlockSpec((2 * hdim, 2 * hdim), full),
            pl.BlockSpec((1, 2 * hdim), full),
            pl.BlockSpec((2 * hdim, 2 * hdim), full),
            pl.BlockSpec((1, 2 * hdim), full),
            pl.BlockSpec((2 * hdim, 16), full),
            pl.BlockSpec((1, 1), full),
            pl.BlockSpec((16, 2 * hdim), full),
            pl.BlockSpec((16, 16), full),
            pl.BlockSpec((hdim, hdim), full),
            pl.BlockSpec((1, hdim), full),
        ],
        out_specs=[
            pl.BlockSpec((tr, hdim), tile),
            pl.BlockSpec((tr, 8), tile),
        ],
        out_shape=[
            jax.ShapeDtypeStruct((n, hdim), _F32),
            jax.ShapeDtypeStruct((n, 8), _F32),
        ],
    )

    update_call = pl.pallas_call(
        functools.partial(_update_kernel, n=n),
        out_shape=[
            jax.ShapeDtypeStruct((n, hdim), _F32),
            jax.ShapeDtypeStruct((n, 8), _F32),
        ],
    )

    head_call = pl.pallas_call(
        functools.partial(_update_head_kernel, n=n),
        out_shape=[
            jax.ShapeDtypeStruct((n, hdim), _F32),
            jax.ShapeDtypeStruct((n, 8), _F32),
        ],
    )

    h_out = None
    vec8 = None
    for b in range(nb):
        wea = We0[b, :hdim]
        web = We0[b, hdim:2 * hdim]
        wsq = We0[b, 2 * hdim]
        zz = jnp.zeros((hdim, hdim), _F32)
        we1bd = jnp.block([[We1[b], zz], [zz, We1[b]]]).astype(jnp.bfloat16)
        wx0bd = jnp.block([[Wx0[b], zz], [zz, Wx0[b]]]).astype(jnp.bfloat16)
        wx1sel = jnp.zeros((2 * hdim, 16), _F32)
        wx1sel = wx1sel.at[:hdim, 0:8].set(jnp.broadcast_to(Wx1[b], (hdim, 8)))
        wx1sel = wx1sel.at[hdim:, 8:16].set(jnp.broadcast_to(Wx1[b], (hdim, 8)))
        wx1sel = wx1sel.astype(jnp.bfloat16)
        selw = jnp.zeros((16, 2 * hdim), _F32)
        selw = selw.at[0:8, :hdim].set(jnp.broadcast_to(wsq[None], (8, hdim)))
        selw = selw.at[8:16, hdim:].set(jnp.broadcast_to(wsq[None], (8, hdim)))
        be1d = jnp.concatenate([be1[b], be1[b]])[None]
        bx0d = jnp.concatenate([bx0[b], bx0[b]])[None]

        amat, bmat, pmat, qmat, xmat = prep_call(x8, h, wea, web, be0[b][None])
        a2 = amat.reshape(n2, 2 * hdim)[None]
        q2 = qmat.reshape(n2, 16)[None]
        x2 = xmat.reshape(n2, 16)[None]
        magg, shift = edge_call(a2, amat, bmat, pmat, q2, x2,
                                gt_const, g_const,
                                we1bd, be1d, wx0bd, bx0d,
                                wx1sel, bx1[b][None], selw, sel16,
                                We1[b], be1[b][None])
        if b < nb - 1:
            h, x8 = update_call(h, x8, magg, shift,
                                Wh0[b, :hdim], Wh0[b, hdim:], bh0[b][None],
                                Wh1[b], bh1[b][None])
        else:
            h_out, vec8 = head_call(h, x8, magg, shift, pos8,
                                    Wh0[b, :hdim], Wh0[b, hdim:],
                                    bh0[b][None], Wh1[b], bh1[b][None],
                                    Wf, bf[None])

    vectors = vec8[:, 0:3][:, None, :]
    return vectors, h_out


# speedup vs baseline: 1.0886x; 1.0886x over previous
import functools

import jax
import jax.numpy as jnp
from jax.experimental import pallas as pl

_F32 = jnp.float32
_TR = 16


def _silu(z):
    t = 0.5 * z
    return t * jnp.tanh(t) + t


def _prep_kernel(x8_ref, h_ref, wea_ref, web_ref, be0_ref,
                 a_ref, b_ref, p_ref, q_ref, x_ref):
    x = x8_ref[...]
    h = h_ref[...]
    n = x.shape[0]
    a_ref[...] = jnp.dot(h, wea_ref[...], preferred_element_type=_F32)
    b_ref[...] = jnp.dot(h, web_ref[...], preferred_element_type=_F32) \
        + be0_ref[...]
    x3 = x[:, 0:3]
    nx = jnp.sum(x3 * x3, axis=1, keepdims=True)
    ones = jnp.ones((n, 1), _F32)
    zer3 = jnp.zeros((n, 3), _F32)
    p_ref[...] = jnp.concatenate([x3, nx, ones, zer3], axis=1)
    q_ref[...] = jnp.concatenate([-2.0 * x3, ones, nx, zer3], axis=1)
    x_ref[...] = jnp.concatenate([x3, ones, jnp.zeros((n, 4), _F32)], axis=1)


def _edge_kernel(a2_ref, a_ref, b_ref, p_ref, q2_ref, x2_ref,
                 gt_ref, g_ref,
                 we1bd_ref, be1d_ref, wx0bd_ref, bx0d_ref,
                 wx1sel_ref, bx1_ref, selw_ref, sel16_ref,
                 we1_ref, be1_ref,
                 magg_ref, shift_ref, *, n, tr, hdim):
    n2 = n // 2
    e2 = tr * n2

    bf16 = jnp.bfloat16

    gt = gt_ref[...]
    g = g_ref[...]

    bt = b_ref[...]
    pt = p_ref[...]
    b2t = jnp.concatenate([bt, bt], axis=1)
    p2t = jnp.concatenate([pt, pt], axis=1)

    p_part = jnp.dot(gt, p2t, preferred_element_type=_F32)
    pq = (q2_ref[...] * p_part.reshape(tr, n2, 16)).reshape(e2, 16)

    sq16 = jnp.maximum(
        jnp.dot(pq, sel16_ref[...], preferred_element_type=_F32), 0.0)
    gtpq = jnp.concatenate([gt, pq], axis=1)
    bsel = jnp.concatenate([b2t, selw_ref[...]], axis=0)
    bsq_part = jnp.dot(gtpq, bsel, preferred_element_type=_F32)

    m0 = _silu(a2_ref[...]
               + bsq_part.astype(bf16).reshape(tr, n2, 2 * hdim)).reshape(
        e2, 2 * hdim)
    m1 = _silu(jnp.dot(m0, we1bd_ref[...],
                       preferred_element_type=_F32).astype(bf16)
               + be1d_ref[...])
    av = _silu(jnp.dot(m1, wx0bd_ref[...],
                       preferred_element_type=_F32).astype(bf16)
               + bx0d_ref[...])
    coef16 = jnp.dot(av, wx1sel_ref[...], preferred_element_type=_F32) \
        + bx1_ref[...]
    scale16 = coef16 / (jnp.sqrt(sq16) + 1.0)

    maggp = jnp.dot(g, m1, preferred_element_type=_F32)
    m0d = _silu(a_ref[...] + bt)
    m1d = _silu(jnp.dot(m0d, we1_ref[...], preferred_element_type=_F32)
                + be1_ref[...])
    magg_ref[...] = maggp[:, :hdim] + maggp[:, hdim:] - m1d

    sx = (scale16.reshape(tr, n2, 16) * x2_ref[...]).astype(bf16)
    sump = jnp.dot(g, sx.reshape(e2, 16), preferred_element_type=_F32)
    part1 = sump[:, 0:8] + sump[:, 8:16]
    ssum = part1[:, 3:4]
    shift_ref[...] = part1 - pt * ssum


def _update_kernel(h_ref, x8_ref, magg_ref, shift_ref,
                   wh0a_ref, wh0b_ref, bh0_ref, wh1_ref, bh1_ref,
                   hn_ref, xn_ref, *, n):
    inv = _F32(1.0 / (n - 1))
    h = h_ref[...]
    magg = magg_ref[...] * inv
    t = _silu(jnp.dot(h, wh0a_ref[...], preferred_element_type=_F32)
              + jnp.dot(magg, wh0b_ref[...], preferred_element_type=_F32)
              + bh0_ref[...])
    dh = jnp.dot(t, wh1_ref[...], preferred_element_type=_F32) + bh1_ref[...]
    hn_ref[...] = h + dh
    xs = x8_ref[:, 0:3] + shift_ref[:, 0:3] * inv
    xn_ref[...] = jnp.concatenate([xs, jnp.zeros((n, 5), _F32)], axis=1)


def _update_head_kernel(h_ref, x8_ref, magg_ref, shift_ref, pos8_ref,
                        wh0a_ref, wh0b_ref, bh0_ref, wh1_ref, bh1_ref,
                        wf_ref, bf_ref,
                        hout_ref, vec_ref, *, n):
    inv = _F32(1.0 / (n - 1))
    h = h_ref[...]
    magg = magg_ref[...] * inv
    t = _silu(jnp.dot(h, wh0a_ref[...], preferred_element_type=_F32)
              + jnp.dot(magg, wh0b_ref[...], preferred_element_type=_F32)
              + bh0_ref[...])
    dh = jnp.dot(t, wh1_ref[...], preferred_element_type=_F32) + bh1_ref[...]
    h2 = h + dh
    xs = x8_ref[:, 0:3] + shift_ref[:, 0:3] * inv
    vec3 = xs - pos8_ref[:, 0:3]
    vec_ref[...] = jnp.concatenate([vec3, jnp.zeros((n, 5), _F32)], axis=1)
    z = h2 - jnp.max(h2, axis=1, keepdims=True)
    ez = jnp.exp(z)
    sm = ez / jnp.sum(ez, axis=1, keepdims=True)
    hout_ref[...] = jnp.dot(sm, wf_ref[...], preferred_element_type=_F32) \
        + bf_ref[...]


def kernel(positions, features, We0, be0, We1, be1, Wx0, bx0, Wx1, bx1,
           Wh0, bh0, Wh1, bh1, Wf, bf):
    n = positions.shape[0]
    hdim = features.shape[-1]
    nb = We0.shape[0]
    tr = _TR
    nt = n // tr
    n2 = n // 2
    e2 = tr * n2

    h = features[:, 0, :].astype(_F32)
    x8 = jnp.pad(positions[:, 0, :].astype(_F32), ((0, 0), (0, 5)))
    pos8 = x8

    recv_of_row = jnp.arange(e2, dtype=jnp.int32) // n2
    gt_const = (recv_of_row[:, None]
                == jnp.arange(tr, dtype=jnp.int32)[None, :]).astype(_F32)
    g_const = gt_const.T.astype(jnp.bfloat16)
    sel16 = ((jnp.arange(16)[:, None] // 8)
             == (jnp.arange(16)[None, :] // 8)).astype(_F32)

    prep_call = pl.pallas_call(
        _prep_kernel,
        out_shape=[
            jax.ShapeDtypeStruct((n, hdim), _F32),
            jax.ShapeDtypeStruct((n, hdim), _F32),
            jax.ShapeDtypeStruct((n, 8), _F32),
            jax.ShapeDtypeStruct((n, 8), _F32),
            jax.ShapeDtypeStruct((n, 8), _F32),
        ],
    )

    full = lambda t: (0, 0)
    tile = lambda t: (t, 0)
    edge_call = pl.pallas_call(
        functools.partial(_edge_kernel, n=n, tr=tr, hdim=hdim),
        grid=(nt,),
        in_specs=[
            pl.BlockSpec((1, n2, 2 * hdim), lambda t: (0, 0, 0)),
            pl.BlockSpec((tr, hdim), tile),
            pl.BlockSpec((tr, hdim), tile),
            pl.BlockSpec((tr, 8), tile),
            pl.BlockSpec((1, n2, 16), lambda t: (0, 0, 0)),
            pl.BlockSpec((1, n2, 16), lambda t: (0, 0, 0)),
            pl.BlockSpec((e2, tr), full),
            pl.BlockSpec((tr, e2), full),
            pl.BlockSpec((2 * hdim, 2 * hdim), full),
            pl.BlockSpec((1, 2 * hdim), full),
            pl.BlockSpec((2 * hdim, 2 * hdim), full),
            pl.BlockSpec((1, 2 * hdim), full),
            pl.BlockSpec((2 * hdim, 16), full),
            pl.BlockSpec((1, 1), full),
            pl.BlockSpec((16, 2 * hdim), full),
            pl.BlockSpec((16, 16), full),
            pl.BlockSpec((hdim, hdim), full),
            pl.BlockSpec((1, hdim), full),
        ],
        out_specs=[
            pl.BlockSpec((tr, hdim), tile),
            pl.BlockSpec((tr, 8), tile),
        ],
        out_shape=[
            jax.ShapeDtypeStruct((n, hdim), _F32),
            jax.ShapeDtypeStruct((n, 8), _F32),
        ],
    )

    update_call = pl.pallas_call(
        functools.partial(_update_kernel, n=n),
        out_shape=[
            jax.ShapeDtypeStruct((n, hdim), _F32),
            jax.ShapeDtypeStruct((n, 8), _F32),
        ],
    )

    head_call = pl.pallas_call(
        functools.partial(_update_head_kernel, n=n),
        out_shape=[
            jax.ShapeDtypeStruct((n, hdim), _F32),
            jax.ShapeDtypeStruct((n, 8), _F32),
        ],
    )

    h_out = None
    vec8 = None
    for b in range(nb):
        wea = We0[b, :hdim]
        web = We0[b, hdim:2 * hdim]
        wsq = We0[b, 2 * hdim]
        zz = jnp.zeros((hdim, hdim), _F32)
        we1bd = jnp.block([[We1[b], zz], [zz, We1[b]]]).astype(jnp.bfloat16)
        wx0bd = jnp.block([[Wx0[b], zz], [zz, Wx0[b]]]).astype(jnp.bfloat16)
        wx1sel = jnp.zeros((2 * hdim, 16), _F32)
        wx1sel = wx1sel.at[:hdim, 0:8].set(jnp.broadcast_to(Wx1[b], (hdim, 8)))
        wx1sel = wx1sel.at[hdim:, 8:16].set(jnp.broadcast_to(Wx1[b], (hdim, 8)))
        wx1sel = wx1sel.astype(jnp.bfloat16)
        selw = jnp.zeros((16, 2 * hdim), _F32)
        selw = selw.at[0:8, :hdim].set(jnp.broadcast_to(wsq[None], (8, hdim)))
        selw = selw.at[8:16, hdim:].set(jnp.broadcast_to(wsq[None], (8, hdim)))
        be1d = jnp.concatenate([be1[b], be1[b]])[None].astype(jnp.bfloat16)
        bx0d = jnp.concatenate([bx0[b], bx0[b]])[None].astype(jnp.bfloat16)

        amat, bmat, pmat, qmat, xmat = prep_call(x8, h, wea, web, be0[b][None])
        a2 = amat.reshape(n2, 2 * hdim)[None].astype(jnp.bfloat16)
        q2 = qmat.reshape(n2, 16)[None]
        x2 = xmat.reshape(n2, 16)[None]
        magg, shift = edge_call(a2, amat, bmat, pmat, q2, x2,
                                gt_const, g_const,
                                we1bd, be1d, wx0bd, bx0d,
                                wx1sel, bx1[b][None], selw, sel16,
                                We1[b], be1[b][None])
        if b < nb - 1:
            h, x8 = update_call(h, x8, magg, shift,
                                Wh0[b, :hdim], Wh0[b, hdim:], bh0[b][None],
                                Wh1[b], bh1[b][None])
        else:
            h_out, vec8 = head_call(h, x8, magg, shift, pos8,
                                    Wh0[b, :hdim], Wh0[b, hdim:],
                                    bh0[b][None], Wh1[b], bh1[b][None],
                                    Wf, bf[None])

    vectors = vec8[:, 0:3][:, None, :]
    return vectors, h_out


# speedup vs baseline: 1.2370x; 1.1363x over previous
import functools

import jax
import jax.numpy as jnp
from jax.experimental import pallas as pl

_F32 = jnp.float32
_TR = 16


def _silu(z):
    t = 0.5 * z
    return t * jnp.tanh(t) + t


def _prep_kernel(x8_ref, h_ref, wea_ref, web_ref, be0_ref,
                 a_ref, b_ref, p_ref, q_ref, x_ref):
    x = x8_ref[...]
    h = h_ref[...]
    n = x.shape[0]
    a_ref[...] = jnp.dot(h, wea_ref[...], preferred_element_type=_F32)
    b_ref[...] = jnp.dot(h, web_ref[...], preferred_element_type=_F32) \
        + be0_ref[...]
    x3 = x[:, 0:3]
    nx = jnp.sum(x3 * x3, axis=1, keepdims=True)
    ones = jnp.ones((n, 1), _F32)
    zer3 = jnp.zeros((n, 3), _F32)
    p_ref[...] = jnp.concatenate([x3, nx, ones, zer3], axis=1)
    q_ref[...] = jnp.concatenate([-2.0 * x3, ones, nx, zer3], axis=1)
    x_ref[...] = jnp.concatenate([x3, ones, jnp.zeros((n, 4), _F32)], axis=1)


def _edge_kernel(a2_ref, a_ref, b_ref, p_ref, q2_ref, x2_ref,
                 gt_ref, g_ref,
                 we1bd_ref, be1d_ref, wx0bd_ref, bx0d_ref,
                 wx1sel_ref, bx1_ref, selw_ref, sel16_ref,
                 we1_ref, be1_ref,
                 magg_ref, shift_ref, *, n, tr, hdim):
    n2 = n // 2
    e2 = tr * n2

    bf16 = jnp.bfloat16

    gt = gt_ref[...]
    g = g_ref[...]

    bt = b_ref[...]
    pt = p_ref[...]
    b2t = jnp.concatenate([bt, bt], axis=1).astype(bf16)
    p2t = jnp.concatenate([pt, pt], axis=1).astype(bf16)

    p_part = jnp.dot(gt, p2t,
                     preferred_element_type=_F32).astype(bf16)
    pq = (q2_ref[...] * p_part.reshape(tr, n2, 16)).reshape(e2, 16)

    sq16 = jnp.maximum(
        jnp.dot(pq, sel16_ref[...], preferred_element_type=_F32).astype(bf16),
        jnp.array(0.0, bf16))
    gtpq = jnp.concatenate([gt, pq], axis=1)
    bsel = jnp.concatenate([b2t, selw_ref[...]], axis=0)
    bsq_part = jnp.dot(gtpq, bsel, preferred_element_type=_F32)

    m0 = _silu(a2_ref[...]
               + bsq_part.astype(bf16).reshape(tr, n2, 2 * hdim)).reshape(
        e2, 2 * hdim)
    m1 = _silu(jnp.dot(m0, we1bd_ref[...],
                       preferred_element_type=_F32).astype(bf16)
               + be1d_ref[...])
    av = _silu(jnp.dot(m1, wx0bd_ref[...],
                       preferred_element_type=_F32).astype(bf16)
               + bx0d_ref[...])
    coef16 = jnp.dot(av, wx1sel_ref[...],
                     preferred_element_type=_F32).astype(bf16) + bx1_ref[...]
    scale16 = coef16 / (jnp.sqrt(sq16) + 1.0)

    maggp = jnp.dot(g, m1, preferred_element_type=_F32)
    m0d = _silu(a_ref[...] + bt)
    m1d = _silu(jnp.dot(m0d, we1_ref[...], preferred_element_type=_F32)
                + be1_ref[...])
    magg_ref[...] = maggp[:, :hdim] + maggp[:, hdim:] - m1d

    sx = scale16.reshape(tr, n2, 16) * x2_ref[...]
    sump = jnp.dot(g, sx.reshape(e2, 16), preferred_element_type=_F32)
    part1 = sump[:, 0:8] + sump[:, 8:16]
    ssum = part1[:, 3:4]
    shift_ref[...] = part1 - pt * ssum


def _update_kernel(h_ref, x8_ref, magg_ref, shift_ref,
                   wh0a_ref, wh0b_ref, bh0_ref, wh1_ref, bh1_ref,
                   hn_ref, xn_ref, *, n):
    inv = _F32(1.0 / (n - 1))
    h = h_ref[...]
    magg = magg_ref[...] * inv
    t = _silu(jnp.dot(h, wh0a_ref[...], preferred_element_type=_F32)
              + jnp.dot(magg, wh0b_ref[...], preferred_element_type=_F32)
              + bh0_ref[...])
    dh = jnp.dot(t, wh1_ref[...], preferred_element_type=_F32) + bh1_ref[...]
    hn_ref[...] = h + dh
    xs = x8_ref[:, 0:3] + shift_ref[:, 0:3] * inv
    xn_ref[...] = jnp.concatenate([xs, jnp.zeros((n, 5), _F32)], axis=1)


def _update_head_kernel(h_ref, x8_ref, magg_ref, shift_ref, pos8_ref,
                        wh0a_ref, wh0b_ref, bh0_ref, wh1_ref, bh1_ref,
                        wf_ref, bf_ref,
                        hout_ref, vec_ref, *, n):
    inv = _F32(1.0 / (n - 1))
    h = h_ref[...]
    magg = magg_ref[...] * inv
    t = _silu(jnp.dot(h, wh0a_ref[...], preferred_element_type=_F32)
              + jnp.dot(magg, wh0b_ref[...], preferred_element_type=_F32)
              + bh0_ref[...])
    dh = jnp.dot(t, wh1_ref[...], preferred_element_type=_F32) + bh1_ref[...]
    h2 = h + dh
    xs = x8_ref[:, 0:3] + shift_ref[:, 0:3] * inv
    vec3 = xs - pos8_ref[:, 0:3]
    vec_ref[...] = jnp.concatenate([vec3, jnp.zeros((n, 5), _F32)], axis=1)
    z = h2 - jnp.max(h2, axis=1, keepdims=True)
    ez = jnp.exp(z)
    sm = ez / jnp.sum(ez, axis=1, keepdims=True)
    hout_ref[...] = jnp.dot(sm, wf_ref[...], preferred_element_type=_F32) \
        + bf_ref[...]


def kernel(positions, features, We0, be0, We1, be1, Wx0, bx0, Wx1, bx1,
           Wh0, bh0, Wh1, bh1, Wf, bf):
    n = positions.shape[0]
    hdim = features.shape[-1]
    nb = We0.shape[0]
    tr = _TR
    nt = n // tr
    n2 = n // 2
    e2 = tr * n2

    h = features[:, 0, :].astype(_F32)
    x8 = jnp.pad(positions[:, 0, :].astype(_F32), ((0, 0), (0, 5)))
    pos8 = x8

    recv_of_row = jnp.arange(e2, dtype=jnp.int32) // n2
    gt_const = (recv_of_row[:, None]
                == jnp.arange(tr, dtype=jnp.int32)[None, :]).astype(
                    jnp.bfloat16)
    g_const = gt_const.T
    sel16 = ((jnp.arange(16)[:, None] // 8)
             == (jnp.arange(16)[None, :] // 8)).astype(jnp.bfloat16)

    prep_call = pl.pallas_call(
        _prep_kernel,
        out_shape=[
            jax.ShapeDtypeStruct((n, hdim), _F32),
            jax.ShapeDtypeStruct((n, hdim), _F32),
            jax.ShapeDtypeStruct((n, 8), _F32),
            jax.ShapeDtypeStruct((n, 8), _F32),
            jax.ShapeDtypeStruct((n, 8), _F32),
        ],
    )

    full = lambda t: (0, 0)
    tile = lambda t: (t, 0)
    edge_call = pl.pallas_call(
        functools.partial(_edge_kernel, n=n, tr=tr, hdim=hdim),
        grid=(nt,),
        in_specs=[
            pl.BlockSpec((1, n2, 2 * hdim), lambda t: (0, 0, 0)),
            pl.BlockSpec((tr, hdim), tile),
            pl.BlockSpec((tr, hdim), tile),
            pl.BlockSpec((tr, 8), tile),
            pl.BlockSpec((1, n2, 16), lambda t: (0, 0, 0)),
            pl.BlockSpec((1, n2, 16), lambda t: (0, 0, 0)),
            pl.BlockSpec((e2, tr), full),
            pl.BlockSpec((tr, e2), full),
            pl.BlockSpec((2 * hdim, 2 * hdim), full),
            pl.BlockSpec((1, 2 * hdim), full),
            pl.BlockSpec((2 * hdim, 2 * hdim), full),
            pl.BlockSpec((1, 2 * hdim), full),
            pl.BlockSpec((2 * hdim, 16), full),
            pl.BlockSpec((1, 1), full),
            pl.BlockSpec((16, 2 * hdim), full),
            pl.BlockSpec((16, 16), full),
            pl.BlockSpec((hdim, hdim), full),
            pl.BlockSpec((1, hdim), full),
        ],
        out_specs=[
            pl.BlockSpec((tr, hdim), tile),
            pl.BlockSpec((tr, 8), tile),
        ],
        out_shape=[
            jax.ShapeDtypeStruct((n, hdim), _F32),
            jax.ShapeDtypeStruct((n, 8), _F32),
        ],
    )

    update_call = pl.pallas_call(
        functools.partial(_update_kernel, n=n),
        out_shape=[
            jax.ShapeDtypeStruct((n, hdim), _F32),
            jax.ShapeDtypeStruct((n, 8), _F32),
        ],
    )

    head_call = pl.pallas_call(
        functools.partial(_update_head_kernel, n=n),
        out_shape=[
            jax.ShapeDtypeStruct((n, hdim), _F32),
            jax.ShapeDtypeStruct((n, 8), _F32),
        ],
    )

    h_out = None
    vec8 = None
    for b in range(nb):
        wea = We0[b, :hdim]
        web = We0[b, hdim:2 * hdim]
        wsq = We0[b, 2 * hdim]
        zz = jnp.zeros((hdim, hdim), _F32)
        we1bd = jnp.block([[We1[b], zz], [zz, We1[b]]]).astype(jnp.bfloat16)
        wx0bd = jnp.block([[Wx0[b], zz], [zz, Wx0[b]]]).astype(jnp.bfloat16)
        wx1sel = jnp.zeros((2 * hdim, 16), _F32)
        wx1sel = wx1sel.at[:hdim, 0:8].set(jnp.broadcast_to(Wx1[b], (hdim, 8)))
        wx1sel = wx1sel.at[hdim:, 8:16].set(jnp.broadcast_to(Wx1[b], (hdim, 8)))
        wx1sel = wx1sel.astype(jnp.bfloat16)
        selw = jnp.zeros((16, 2 * hdim), _F32)
        selw = selw.at[0:8, :hdim].set(jnp.broadcast_to(wsq[None], (8, hdim)))
        selw = selw.at[8:16, hdim:].set(jnp.broadcast_to(wsq[None], (8, hdim)))
        selw = selw.astype(jnp.bfloat16)
        be1d = jnp.concatenate([be1[b], be1[b]])[None].astype(jnp.bfloat16)
        bx0d = jnp.concatenate([bx0[b], bx0[b]])[None].astype(jnp.bfloat16)

        amat, bmat, pmat, qmat, xmat = prep_call(x8, h, wea, web, be0[b][None])
        a2 = amat.reshape(n2, 2 * hdim)[None].astype(jnp.bfloat16)
        q2 = qmat.reshape(n2, 16)[None].astype(jnp.bfloat16)
        x2 = xmat.reshape(n2, 16)[None].astype(jnp.bfloat16)
        magg, shift = edge_call(a2, amat, bmat, pmat, q2, x2,
                                gt_const, g_const,
                                we1bd, be1d, wx0bd, bx0d,
                                wx1sel, bx1[b][None].astype(jnp.bfloat16),
                                selw, sel16,
                                We1[b], be1[b][None])
        if b < nb - 1:
            h, x8 = update_call(h, x8, magg, shift,
                                Wh0[b, :hdim], Wh0[b, hdim:], bh0[b][None],
                                Wh1[b], bh1[b][None])
        else:
            h_out, vec8 = head_call(h, x8, magg, shift, pos8,
                                    Wh0[b, :hdim], Wh0[b, hdim:],
                                    bh0[b][None], Wh1[b], bh1[b][None],
                                    Wf, bf[None])

    vectors = vec8[:, 0:3][:, None, :]
    return vectors, h_out


# speedup vs baseline: 1.3779x; 1.1139x over previous
import functools

import jax
import jax.numpy as jnp
from jax.experimental import pallas as pl

_F32 = jnp.float32
_TR = 16


def _silu(z):
    t = 0.5 * z
    return t * jnp.tanh(t) + t


def _prep_kernel(x8_ref, h_ref, wea_ref, web_ref, be0_ref,
                 a_ref, b_ref, p_ref, q_ref, x_ref):
    x = x8_ref[...]
    h = h_ref[...]
    n = x.shape[0]
    a_ref[...] = jnp.dot(h, wea_ref[...], preferred_element_type=_F32)
    b_ref[...] = jnp.dot(h, web_ref[...], preferred_element_type=_F32) \
        + be0_ref[...]
    x3 = x[:, 0:3]
    nx = jnp.sum(x3 * x3, axis=1, keepdims=True)
    ones = jnp.ones((n, 1), _F32)
    zer3 = jnp.zeros((n, 3), _F32)
    p_ref[...] = jnp.concatenate([x3, nx, ones, zer3], axis=1)
    q_ref[...] = jnp.concatenate([-2.0 * x3, ones, nx, zer3], axis=1)
    x_ref[...] = jnp.concatenate([x3, ones, jnp.zeros((n, 4), _F32)], axis=1)


def _edge_kernel(a2_ref, a_ref, b_ref, p_ref, q2_ref, x2_ref,
                 g_ref,
                 we1bd_ref, be1d_ref, wx0bd_ref, bx0d_ref,
                 wx1sel_ref, bx1_ref, selw_ref, sel16_ref,
                 we1_ref, be1_ref,
                 magg_ref, shift_ref, *, n, tr, hdim):
    n2 = n // 2
    e2 = tr * n2

    bf16 = jnp.bfloat16

    g = g_ref[...]

    bt = b_ref[...]
    pt = p_ref[...]
    b2t = jnp.concatenate([bt, bt], axis=1).astype(bf16)
    p2t = jnp.concatenate([pt, pt], axis=1).astype(bf16)

    pq = (q2_ref[...] * p2t[:, None, :]).reshape(e2, 16)

    sq16 = jnp.maximum(
        jnp.dot(pq, sel16_ref[...], preferred_element_type=_F32).astype(bf16),
        jnp.array(1e-12, bf16))
    sqw = jnp.dot(pq, selw_ref[...], preferred_element_type=_F32)

    m0 = _silu(a2_ref[...] + b2t[:, None, :]
               + sqw.astype(bf16).reshape(tr, n2, 2 * hdim)).reshape(
        e2, 2 * hdim)
    m1 = _silu(jnp.dot(m0, we1bd_ref[...],
                       preferred_element_type=_F32).astype(bf16)
               + be1d_ref[...])
    av = _silu(jnp.dot(m1, wx0bd_ref[...],
                       preferred_element_type=_F32).astype(bf16)
               + bx0d_ref[...])
    coef16 = jnp.dot(av, wx1sel_ref[...],
                     preferred_element_type=_F32).astype(bf16) + bx1_ref[...]
    den = 1.0 + sq16 * jax.lax.rsqrt(sq16)
    scale16 = coef16 * jax.lax.reciprocal(den)

    maggp = jnp.dot(g, m1, preferred_element_type=_F32)
    m0d = _silu(a_ref[...] + bt)
    m1d = _silu(jnp.dot(m0d, we1_ref[...], preferred_element_type=_F32)
                + be1_ref[...])
    magg_ref[...] = maggp[:, :hdim] + maggp[:, hdim:] - m1d

    sx = scale16.reshape(tr, n2, 16) * x2_ref[...]
    sump = jnp.dot(g, sx.reshape(e2, 16), preferred_element_type=_F32)
    part1 = sump[:, 0:8] + sump[:, 8:16]
    ssum = part1[:, 3:4]
    shift_ref[...] = part1 - pt * ssum


def _update_kernel(h_ref, x8_ref, magg_ref, shift_ref,
                   wh0a_ref, wh0b_ref, bh0_ref, wh1_ref, bh1_ref,
                   hn_ref, xn_ref, *, n):
    inv = _F32(1.0 / (n - 1))
    h = h_ref[...]
    magg = magg_ref[...] * inv
    t = _silu(jnp.dot(h, wh0a_ref[...], preferred_element_type=_F32)
              + jnp.dot(magg, wh0b_ref[...], preferred_element_type=_F32)
              + bh0_ref[...])
    dh = jnp.dot(t, wh1_ref[...], preferred_element_type=_F32) + bh1_ref[...]
    hn_ref[...] = h + dh
    xs = x8_ref[:, 0:3] + shift_ref[:, 0:3] * inv
    xn_ref[...] = jnp.concatenate([xs, jnp.zeros((n, 5), _F32)], axis=1)


def _update_head_kernel(h_ref, x8_ref, magg_ref, shift_ref, pos8_ref,
                        wh0a_ref, wh0b_ref, bh0_ref, wh1_ref, bh1_ref,
                        wf_ref, bf_ref,
                        hout_ref, vec_ref, *, n):
    inv = _F32(1.0 / (n - 1))
    h = h_ref[...]
    magg = magg_ref[...] * inv
    t = _silu(jnp.dot(h, wh0a_ref[...], preferred_element_type=_F32)
              + jnp.dot(magg, wh0b_ref[...], preferred_element_type=_F32)
              + bh0_ref[...])
    dh = jnp.dot(t, wh1_ref[...], preferred_element_type=_F32) + bh1_ref[...]
    h2 = h + dh
    xs = x8_ref[:, 0:3] + shift_ref[:, 0:3] * inv
    vec3 = xs - pos8_ref[:, 0:3]
    vec_ref[...] = jnp.concatenate([vec3, jnp.zeros((n, 5), _F32)], axis=1)
    z = h2 - jnp.max(h2, axis=1, keepdims=True)
    ez = jnp.exp(z)
    sm = ez / jnp.sum(ez, axis=1, keepdims=True)
    hout_ref[...] = jnp.dot(sm, wf_ref[...], preferred_element_type=_F32) \
        + bf_ref[...]


def kernel(positions, features, We0, be0, We1, be1, Wx0, bx0, Wx1, bx1,
           Wh0, bh0, Wh1, bh1, Wf, bf):
    n = positions.shape[0]
    hdim = features.shape[-1]
    nb = We0.shape[0]
    tr = _TR
    nt = n // tr
    n2 = n // 2
    e2 = tr * n2

    h = features[:, 0, :].astype(_F32)
    x8 = jnp.pad(positions[:, 0, :].astype(_F32), ((0, 0), (0, 5)))
    pos8 = x8

    recv_of_row = jnp.arange(e2, dtype=jnp.int32) // n2
    g_const = (recv_of_row[None, :]
               == jnp.arange(tr, dtype=jnp.int32)[:, None]).astype(
                   jnp.bfloat16)
    sel16 = ((jnp.arange(16)[:, None] // 8)
             == (jnp.arange(16)[None, :] // 8)).astype(jnp.bfloat16)

    prep_call = pl.pallas_call(
        _prep_kernel,
        out_shape=[
            jax.ShapeDtypeStruct((n, hdim), _F32),
            jax.ShapeDtypeStruct((n, hdim), _F32),
            jax.ShapeDtypeStruct((n, 8), _F32),
            jax.ShapeDtypeStruct((n, 8), _F32),
            jax.ShapeDtypeStruct((n, 8), _F32),
        ],
    )

    full = lambda t: (0, 0)
    tile = lambda t: (t, 0)
    edge_call = pl.pallas_call(
        functools.partial(_edge_kernel, n=n, tr=tr, hdim=hdim),
        grid=(nt,),
        in_specs=[
            pl.BlockSpec((1, n2, 2 * hdim), lambda t: (0, 0, 0)),
            pl.BlockSpec((tr, hdim), tile),
            pl.BlockSpec((tr, hdim), tile),
            pl.BlockSpec((tr, 8), tile),
            pl.BlockSpec((1, n2, 16), lambda t: (0, 0, 0)),
            pl.BlockSpec((1, n2, 16), lambda t: (0, 0, 0)),
            pl.BlockSpec((tr, e2), full),
            pl.BlockSpec((2 * hdim, 2 * hdim), full),
            pl.BlockSpec((1, 2 * hdim), full),
            pl.BlockSpec((2 * hdim, 2 * hdim), full),
            pl.BlockSpec((1, 2 * hdim), full),
            pl.BlockSpec((2 * hdim, 16), full),
            pl.BlockSpec((1, 1), full),
            pl.BlockSpec((16, 2 * hdim), full),
            pl.BlockSpec((16, 16), full),
            pl.BlockSpec((hdim, hdim), full),
            pl.BlockSpec((1, hdim), full),
        ],
        out_specs=[
            pl.BlockSpec((tr, hdim), tile),
            pl.BlockSpec((tr, 8), tile),
        ],
        out_shape=[
            jax.ShapeDtypeStruct((n, hdim), _F32),
            jax.ShapeDtypeStruct((n, 8), _F32),
        ],
    )

    update_call = pl.pallas_call(
        functools.partial(_update_kernel, n=n),
        out_shape=[
            jax.ShapeDtypeStruct((n, hdim), _F32),
            jax.ShapeDtypeStruct((n, 8), _F32),
        ],
    )

    head_call = pl.pallas_call(
        functools.partial(_update_head_kernel, n=n),
        out_shape=[
            jax.ShapeDtypeStruct((n, hdim), _F32),
            jax.ShapeDtypeStruct((n, 8), _F32),
        ],
    )

    h_out = None
    vec8 = None
    for b in range(nb):
        wea = We0[b, :hdim]
        web = We0[b, hdim:2 * hdim]
        wsq = We0[b, 2 * hdim]
        zz = jnp.zeros((hdim, hdim), _F32)
        we1bd = jnp.block([[We1[b], zz], [zz, We1[b]]]).astype(jnp.bfloat16)
        wx0bd = jnp.block([[Wx0[b], zz], [zz, Wx0[b]]]).astype(jnp.bfloat16)
        wx1sel = jnp.zeros((2 * hdim, 16), _F32)
        wx1sel = wx1sel.at[:hdim, 0:8].set(jnp.broadcast_to(Wx1[b], (hdim, 8)))
        wx1sel = wx1sel.at[hdim:, 8:16].set(jnp.broadcast_to(Wx1[b], (hdim, 8)))
        wx1sel = wx1sel.astype(jnp.bfloat16)
        selw = jnp.zeros((16, 2 * hdim), _F32)
        selw = selw.at[0:8, :hdim].set(jnp.broadcast_to(wsq[None], (8, hdim)))
        selw = selw.at[8:16, hdim:].set(jnp.broadcast_to(wsq[None], (8, hdim)))
        selw = selw.astype(jnp.bfloat16)
        be1d = jnp.concatenate([be1[b], be1[b]])[None].astype(jnp.bfloat16)
        bx0d = jnp.concatenate([bx0[b], bx0[b]])[None].astype(jnp.bfloat16)

        amat, bmat, pmat, qmat, xmat = prep_call(x8, h, wea, web, be0[b][None])
        a2 = amat.reshape(n2, 2 * hdim)[None].astype(jnp.bfloat16)
        q2 = qmat.reshape(n2, 16)[None].astype(jnp.bfloat16)
        x2 = xmat.reshape(n2, 16)[None].astype(jnp.bfloat16)
        magg, shift = edge_call(a2, amat, bmat, pmat, q2, x2,
                                g_const,
                                we1bd, be1d, wx0bd, bx0d,
                                wx1sel, bx1[b][None].astype(jnp.bfloat16),
                                selw, sel16,
                                We1[b], be1[b][None])
        if b < nb - 1:
            h, x8 = update_call(h, x8, magg, shift,
                                Wh0[b, :hdim], Wh0[b, hdim:], bh0[b][None],
                                Wh1[b], bh1[b][None])
        else:
            h_out, vec8 = head_call(h, x8, magg, shift, pos8,
                                    Wh0[b, :hdim], Wh0[b, hdim:],
                                    bh0[b][None], Wh1[b], bh1[b][None],
                                    Wf, bf[None])

    vectors = vec8[:, 0:3][:, None, :]
    return vectors, h_out


# speedup vs baseline: 1.3963x; 1.0133x over previous
import functools

import jax
import jax.numpy as jnp
from jax.experimental import pallas as pl

_F32 = jnp.float32
_BF16 = jnp.bfloat16
_TR = 16


def _silu(z):
    t = 0.5 * z
    return t * jnp.tanh(t) + t


def _wprep_kernel(we0_ref, we1_ref, wx0_ref, wx1_ref, be1_ref, bx0_ref,
                  bx1_ref,
                  wea_ref, web_ref, we1bd_ref, wx0bd_ref, wx1sel_ref,
                  selw_ref, be1d_ref, bx0d_ref, bx1d_ref, *, hdim):
    nb = we0_ref.shape[0]
    we0 = we0_ref[...]
    wea_ref[...] = we0[:, :hdim, :]
    web_ref[...] = we0[:, hdim:2 * hdim, :]

    def blockdiag(w):
        z = jnp.zeros((nb, hdim, hdim), _F32)
        top = jnp.concatenate([w, z], axis=2)
        bot = jnp.concatenate([z, w], axis=2)
        return jnp.concatenate([top, bot], axis=1).astype(_BF16)

    we1bd_ref[...] = blockdiag(we1_ref[...])
    wx0bd_ref[...] = blockdiag(wx0_ref[...])

    w8 = jnp.broadcast_to(wx1_ref[...], (nb, hdim, 8))
    z8 = jnp.zeros((nb, hdim, 8), _F32)
    top = jnp.concatenate([w8, z8], axis=2)
    bot = jnp.concatenate([z8, w8], axis=2)
    wx1sel_ref[...] = jnp.concatenate([top, bot], axis=1).astype(_BF16)

    wsq = jnp.broadcast_to(we0[:, 2 * hdim:2 * hdim + 1, :], (nb, 8, hdim))
    zs = jnp.zeros((nb, 8, hdim), _F32)
    r1 = jnp.concatenate([wsq, zs], axis=2)
    r2 = jnp.concatenate([zs, wsq], axis=2)
    selw_ref[...] = jnp.concatenate([r1, r2], axis=1).astype(_BF16)

    be1 = be1_ref[...]
    bx0 = bx0_ref[...]
    be1d_ref[...] = jnp.concatenate([be1, be1], axis=1).astype(_BF16)
    bx0d_ref[...] = jnp.concatenate([bx0, bx0], axis=1).astype(_BF16)
    bx1d_ref[...] = bx1_ref[...].astype(_BF16)


def _node_prep(x3, h, wea, web, be0r,
               a_ref, ab_ref, b_ref, p_ref, q_ref, x_ref):
    n = x3.shape[0]
    a = jnp.dot(h, wea, preferred_element_type=_F32)
    a_ref[...] = a
    ab_ref[...] = a.astype(_BF16)
    b_ref[...] = jnp.dot(h, web, preferred_element_type=_F32) + be0r
    nx = jnp.sum(x3 * x3, axis=1, keepdims=True)
    ones = jnp.ones((n, 1), _F32)
    zer3 = jnp.zeros((n, 3), _F32)
    p_ref[...] = jnp.concatenate([x3, nx, ones, zer3], axis=1)
    q_ref[...] = jnp.concatenate(
        [-2.0 * x3, ones, nx, zer3], axis=1).astype(_BF16)
    x_ref[...] = jnp.concatenate(
        [x3, ones, jnp.zeros((n, 4), _F32)], axis=1).astype(_BF16)


def _prep_kernel(xr_ref, h_ref, wea_ref, web_ref, be0_ref,
                 a_ref, ab_ref, b_ref, p_ref, q_ref, x_ref):
    _node_prep(xr_ref[...], h_ref[...], wea_ref[...], web_ref[...],
               be0_ref[...], a_ref, ab_ref, b_ref, p_ref, q_ref, x_ref)


def _edge_kernel(a2_ref, a_ref, b_ref, p_ref, q2_ref, x2_ref,
                 g_ref,
                 we1bd_ref, be1d_ref, wx0bd_ref, bx0d_ref,
                 wx1sel_ref, bx1_ref, selw_ref, sel16_ref,
                 we1_ref, be1_ref,
                 magg_ref, shift_ref, *, n, tr, hdim):
    n2 = n // 2
    e2 = tr * n2

    g = g_ref[...]

    bt = b_ref[...]
    pt = p_ref[...]
    b2t = jnp.concatenate([bt, bt], axis=1).astype(_BF16)
    p2t = jnp.concatenate([pt, pt], axis=1).astype(_BF16)

    pq = (q2_ref[...] * p2t[:, None, :]).reshape(e2, 16)

    sq16 = jnp.maximum(
        jnp.dot(pq, sel16_ref[...],
                preferred_element_type=_F32).astype(_BF16),
        jnp.array(1e-12, _BF16))
    sqw = jnp.dot(pq, selw_ref[...], preferred_element_type=_F32)

    m0 = _silu(a2_ref[...] + b2t[:, None, :]
               + sqw.astype(_BF16).reshape(tr, n2, 2 * hdim)).reshape(
        e2, 2 * hdim)
    m1 = _silu(jnp.dot(m0, we1bd_ref[...],
                       preferred_element_type=_F32).astype(_BF16)
               + be1d_ref[...])
    av = _silu(jnp.dot(m1, wx0bd_ref[...],
                       preferred_element_type=_F32).astype(_BF16)
               + bx0d_ref[...])
    coef16 = jnp.dot(av, wx1sel_ref[...],
                     preferred_element_type=_F32).astype(_BF16) + bx1_ref[...]
    den = 1.0 + sq16 * jax.lax.rsqrt(sq16)
    scale16 = coef16 * jax.lax.reciprocal(den)

    maggp = jnp.dot(g, m1, preferred_element_type=_F32)
    m0d = _silu(a_ref[...] + bt)
    m1d = _silu(jnp.dot(m0d, we1_ref[...], preferred_element_type=_F32)
                + be1_ref[...])
    magg_ref[...] = maggp[:, :hdim] + maggp[:, hdim:] - m1d

    sx = scale16.reshape(tr, n2, 16) * x2_ref[...]
    sump = jnp.dot(g, sx.reshape(e2, 16), preferred_element_type=_F32)
    part1 = sump[:, 0:8] + sump[:, 8:16]
    ssum = part1[:, 3:4]
    shift_ref[...] = part1 - pt * ssum


def _node_update(h_ref, xr_ref, magg_ref, shift_ref,
                 wh0a_ref, wh0b_ref, bh0_ref, wh1_ref, bh1_ref, n):
    inv = _F32(1.0 / (n - 1))
    h = h_ref[...]
    magg = magg_ref[...] * inv
    t = _silu(jnp.dot(h, wh0a_ref[...], preferred_element_type=_F32)
              + jnp.dot(magg, wh0b_ref[...], preferred_element_type=_F32)
              + bh0_ref[...])
    dh = jnp.dot(t, wh1_ref[...], preferred_element_type=_F32) + bh1_ref[...]
    xs = xr_ref[...] + shift_ref[:, 0:3] * inv
    return h + dh, xs


def _update_prep_kernel(h_ref, xr_ref, magg_ref, shift_ref,
                        wh0a_ref, wh0b_ref, bh0_ref, wh1_ref, bh1_ref,
                        wea_ref, web_ref, be0_ref,
                        hn_ref, xn_ref,
                        a_ref, ab_ref, b_ref, p_ref, q_ref, x_ref, *, n):
    hn, xn = _node_update(h_ref, xr_ref, magg_ref, shift_ref,
                          wh0a_ref, wh0b_ref, bh0_ref, wh1_ref, bh1_ref, n)
    hn_ref[...] = hn
    xn_ref[...] = xn
    _node_prep(xn, hn, wea_ref[...], web_ref[...], be0_ref[...],
               a_ref, ab_ref, b_ref, p_ref, q_ref, x_ref)


def _update_head_kernel(h_ref, xr_ref, magg_ref, shift_ref, pos_ref,
                        wh0a_ref, wh0b_ref, bh0_ref, wh1_ref, bh1_ref,
                        wf_ref, bf_ref,
                        hout_ref, vec_ref, *, n):
    h2, xs = _node_update(h_ref, xr_ref, magg_ref, shift_ref,
                          wh0a_ref, wh0b_ref, bh0_ref, wh1_ref, bh1_ref, n)
    vec_ref[...] = xs - pos_ref[...]
    z = h2 - jnp.max(h2, axis=1, keepdims=True)
    ez = jnp.exp(z)
    sm = ez / jnp.sum(ez, axis=1, keepdims=True)
    hout_ref[...] = jnp.dot(sm, wf_ref[...], preferred_element_type=_F32) \
        + bf_ref[...]


def kernel(positions, features, We0, be0, We1, be1, Wx0, bx0, Wx1, bx1,
           Wh0, bh0, Wh1, bh1, Wf, bf):
    n = positions.shape[0]
    hdim = features.shape[-1]
    nb = We0.shape[0]
    tr = _TR
    nt = n // tr
    n2 = n // 2
    e2 = tr * n2
    h2d = 2 * hdim

    h0 = features[:, 0, :].astype(_F32)
    xr = positions[:, 0, :].astype(_F32)

    recv_of_row = jnp.arange(e2, dtype=jnp.int32) // n2
    g_const = (recv_of_row[None, :]
               == jnp.arange(tr, dtype=jnp.int32)[:, None]).astype(_BF16)
    sel16 = ((jnp.arange(16)[:, None] // 8)
             == (jnp.arange(16)[None, :] // 8)).astype(_BF16)

    wprep_call = pl.pallas_call(
        functools.partial(_wprep_kernel, hdim=hdim),
        out_shape=[
            jax.ShapeDtypeStruct((nb, hdim, hdim), _F32),
            jax.ShapeDtypeStruct((nb, hdim, hdim), _F32),
            jax.ShapeDtypeStruct((nb, h2d, h2d), _BF16),
            jax.ShapeDtypeStruct((nb, h2d, h2d), _BF16),
            jax.ShapeDtypeStruct((nb, h2d, 16), _BF16),
            jax.ShapeDtypeStruct((nb, 16, h2d), _BF16),
            jax.ShapeDtypeStruct((nb, h2d), _BF16),
            jax.ShapeDtypeStruct((nb, h2d), _BF16),
            jax.ShapeDtypeStruct((nb, 1), _BF16),
        ],
    )

    node_out_shape = [
        jax.ShapeDtypeStruct((n, hdim), _F32),
        jax.ShapeDtypeStruct((n, hdim), _BF16),
        jax.ShapeDtypeStruct((n, hdim), _F32),
        jax.ShapeDtypeStruct((n, 8), _F32),
        jax.ShapeDtypeStruct((n, 8), _BF16),
        jax.ShapeDtypeStruct((n, 8), _BF16),
    ]
    prep_call = pl.pallas_call(_prep_kernel, out_shape=node_out_shape)

    full = lambda t: (0, 0)
    tile = lambda t: (t, 0)
    edge_call = pl.pallas_call(
        functools.partial(_edge_kernel, n=n, tr=tr, hdim=hdim),
        grid=(nt,),
        in_specs=[
            pl.BlockSpec((1, n2, h2d), lambda t: (0, 0, 0)),
            pl.BlockSpec((tr, hdim), tile),
            pl.BlockSpec((tr, hdim), tile),
            pl.BlockSpec((tr, 8), tile),
            pl.BlockSpec((1, n2, 16), lambda t: (0, 0, 0)),
            pl.BlockSpec((1, n2, 16), lambda t: (0, 0, 0)),
            pl.BlockSpec((tr, e2), full),
            pl.BlockSpec((h2d, h2d), full),
            pl.BlockSpec((1, h2d), full),
            pl.BlockSpec((h2d, h2d), full),
            pl.BlockSpec((1, h2d), full),
            pl.BlockSpec((h2d, 16), full),
            pl.BlockSpec((1, 1), full),
            pl.BlockSpec((16, h2d), full),
            pl.BlockSpec((16, 16), full),
            pl.BlockSpec((hdim, hdim), full),
            pl.BlockSpec((1, hdim), full),
        ],
        out_specs=[
            pl.BlockSpec((tr, hdim), tile),
            pl.BlockSpec((tr, 8), tile),
        ],
        out_shape=[
            jax.ShapeDtypeStruct((n, hdim), _F32),
            jax.ShapeDtypeStruct((n, 8), _F32),
        ],
    )

    upprep_call = pl.pallas_call(
        functools.partial(_update_prep_kernel, n=n),
        out_shape=[
            jax.ShapeDtypeStruct((n, hdim), _F32),
            jax.ShapeDtypeStruct((n, 3), _F32),
        ] + node_out_shape,
    )

    head_call = pl.pallas_call(
        functools.partial(_update_head_kernel, n=n),
        out_shape=[
            jax.ShapeDtypeStruct((n, hdim), _F32),
            jax.ShapeDtypeStruct((n, 3), _F32),
        ],
    )

    (wea_s, web_s, we1bd_s, wx0bd_s, wx1sel_s, selw_s, be1d_s, bx0d_s,
     bx1d_s) = wprep_call(We0, We1, Wx0, Wx1, be1, bx0, bx1)

    def run_edge(b, nodes):
        amat, ab, bmat, pmat, qmat, xmat = nodes
        return edge_call(ab.reshape(n2, h2d)[None], amat, bmat, pmat,
                         qmat.reshape(n2, 16)[None],
                         xmat.reshape(n2, 16)[None],
                         g_const,
                         we1bd_s[b], be1d_s[b][None], wx0bd_s[b],
                         bx0d_s[b][None], wx1sel_s[b], bx1d_s[b][None],
                         selw_s[b], sel16, We1[b], be1[b][None])

    nodes = prep_call(xr, h0, wea_s[0], web_s[0], be0[0][None])
    hcur, xcur = h0, xr
    for b in range(nb - 1):
        magg, shift = run_edge(b, nodes)
        out = upprep_call(hcur, xcur, magg, shift,
                          Wh0[b][:hdim], Wh0[b][hdim:], bh0[b][None],
                          Wh1[b], bh1[b][None],
                          wea_s[b + 1], web_s[b + 1], be0[b + 1][None])
        hcur, xcur = out[0], out[1]
        nodes = out[2:]

    magg, shift = run_edge(nb - 1, nodes)
    h_out, vec = head_call(hcur, xcur, magg, shift, xr,
                           Wh0[nb - 1][:hdim], Wh0[nb - 1][hdim:],
                           bh0[nb - 1][None], Wh1[nb - 1], bh1[nb - 1][None],
                           Wf, bf[None])

    return vec[:, None, :], h_out
